# baseline (speedup 1.0000x reference)
import jax
import jax.numpy as jnp
from jax.experimental import pallas as pl
from jax.experimental.pallas import tpu as pltpu

_BM = 200


def _gf_kernel(alpha_ref, m_ref, x_ref, h_ref, ah_ref):
    h = jax.lax.dot_general(
        m_ref[...].astype(jnp.bfloat16),
        x_ref[...].astype(jnp.bfloat16),
        dimension_numbers=(((1,), (0,)), ((), ())),
        preferred_element_type=jnp.float32,
    )
    h_ref[...] = h
    ah_ref[...] = alpha_ref[0] * h


def kernel(inp, M, alpha):
    n, k = M.shape
    d = inp.shape[1]
    out = pl.pallas_call(
        _gf_kernel,
        grid=(n // _BM,),
        in_specs=[
            pl.BlockSpec(memory_space=pltpu.SMEM),
            pl.BlockSpec((_BM, k), lambda i: (i, 0)),
            pl.BlockSpec((k, d), lambda i: (0, 0)),
        ],
        out_specs=[
            pl.BlockSpec((_BM, d), lambda i: (i, 0)),
            pl.BlockSpec((_BM, d), lambda i: (i, 0)),
        ],
        out_shape=[
            jax.ShapeDtypeStruct((n, d), jnp.float32),
            jax.ShapeDtypeStruct((n, d), jnp.float32),
        ],
    )(alpha, M, inp)
    return (out[0], out[1])


# speedup vs baseline: 1.0068x; 1.0068x over previous
import jax
import jax.numpy as jnp
from jax.experimental import pallas as pl
from jax.experimental.pallas import tpu as pltpu

_BM = 400
_H = _BM // 2


def _gf_kernel(alpha_ref, m0_ref, m1_ref, x_ref, h_ref, ah_ref):
    x = x_ref[...]
    h0 = jax.lax.dot_general(
        m0_ref[...], x,
        dimension_numbers=(((1,), (0,)), ((), ())),
        preferred_element_type=jnp.float32,
    )
    h1 = jax.lax.dot_general(
        m1_ref[...], x,
        dimension_numbers=(((1,), (0,)), ((), ())),
        preferred_element_type=jnp.float32,
    )
    a = alpha_ref[0]
    h_ref[0:_H, :] = h0
    h_ref[_H:_BM, :] = h1
    ah_ref[0:_H, :] = a * h0
    ah_ref[_H:_BM, :] = a * h1


def kernel(inp, M, alpha):
    n, k = M.shape
    d = inp.shape[1]
    out = pl.pallas_call(
        _gf_kernel,
        grid=(n // _BM,),
        in_specs=[
            pl.BlockSpec(memory_space=pltpu.SMEM),
            pl.BlockSpec((_H, k), lambda i: (2 * i, 0)),
            pl.BlockSpec((_H, k), lambda i: (2 * i + 1, 0)),
            pl.BlockSpec((k, d), lambda i: (0, 0)),
        ],
        out_specs=[
            pl.BlockSpec((_BM, d), lambda i: (i, 0)),
            pl.BlockSpec((_BM, d), lambda i: (i, 0)),
        ],
        out_shape=[
            jax.ShapeDtypeStruct((n, d), jnp.float32),
            jax.ShapeDtypeStruct((n, d), jnp.float32),
        ],
    )(alpha, M, M, inp)
    return (out[0], out[1])
